# baseline (speedup 1.0000x reference)
import functools

import jax
import jax.numpy as jnp
from jax import lax
from jax.experimental import pallas as pl
from jax.experimental.pallas import tpu as pltpu
from jax.experimental.pallas import tpu_sc as plsc

_NC = 2
_NS = 16
_K = 128


@functools.partial(jax.jit, static_argnames=("n_nodes", "channels"))
def _spmm_partials(h, idx2d, zeros, *, n_nodes, channels):
    nrows_total = idx2d.shape[0] // 2
    np_nodes = zeros.shape[0]
    ntiles = _NC * _NS
    nct = nrows_total // ntiles
    rpt = np_nodes // _NS
    mesh = plsc.VectorSubcoreMesh(core_axis_name="c", subcore_axis_name="s")

    nbuf = 2
    nh = 2
    nch = nct // nh
    @functools.partial(
        pl.kernel,
        out_type=jax.ShapeDtypeStruct((2 * np_nodes, channels), jnp.float32),
        mesh=mesh,
        scratch_types=(
            [pltpu.VMEM((nch, _K), jnp.int32)] * 2
            + [pltpu.VMEM((_K, channels), jnp.float32)] * nbuf
            + [pltpu.VMEM_SHARED((np_nodes, channels), jnp.float32)]
            + [pltpu.SemaphoreType.DMA] * (nbuf + 1)
        ),
    )
    def spmm(h_hbm, idx_hbm, zeros_hbm, out_hbm, src_v, dst_v, *scratch):
        rows = scratch[:nbuf]
        acc = scratch[nbuf]
        sem_g = scratch[nbuf + 1:nbuf + 1 + nbuf]
        sem_s = scratch[nbuf + 1 + nbuf]
        cid = lax.axis_index("c")
        sid = lax.axis_index("s")
        tid = cid * _NS + sid

        def gather(j):
            pltpu.async_copy(h_hbm.at[src_v.at[j]], rows[j % nbuf],
                             sem_g[j % nbuf])

        def gwait(j):
            pltpu.make_async_copy(h_hbm.at[src_v.at[j]], rows[j % nbuf],
                                  sem_g[j % nbuf]).wait()

        def scat(j):
            pltpu.async_copy(rows[j % nbuf], acc.at[dst_v.at[j]], sem_s,
                             add=True)

        def swait(j):
            pltpu.make_async_copy(rows[j % nbuf], acc.at[dst_v.at[j]],
                                  sem_s).wait()

        for h in range(nh):
            base = tid * nct + h * nch
            pltpu.sync_copy(idx_hbm.at[pl.ds(base, nch)], src_v)
            pltpu.sync_copy(idx_hbm.at[pl.ds(nrows_total + base, nch)], dst_v)
            for b in range(min(nbuf, nch)):
                gather(b)
            if h == 0:
                pltpu.sync_copy(zeros_hbm.at[pl.ds(sid * rpt, rpt)],
                                acc.at[pl.ds(sid * rpt, rpt)])
                plsc.subcore_barrier()
            for j in range(nch):
                gwait(j)
                if j > 0:
                    swait(j - 1)
                    if j - 1 + nbuf < nch:
                        gather(j - 1 + nbuf)
                scat(j)
            swait(nch - 1)

        plsc.subcore_barrier()
        pltpu.sync_copy(acc.at[pl.ds(sid * rpt, rpt)],
                        out_hbm.at[pl.ds(cid * np_nodes + sid * rpt, rpt)])

    return spmm(h, idx2d, zeros)


def _mm_body(a, w, o):
    o[...] = jnp.dot(a[...], w[...], preferred_element_type=jnp.float32)


def _l1_body(pa0, pa1, pb0, pb1, w1a, w1b, o):
    ha = jnp.maximum(pa0[...] + pa1[...], 0.0)
    hb = jnp.maximum(pb0[...] + pb1[...], 0.0)
    o[...] = (jnp.dot(ha, w1a[...], preferred_element_type=jnp.float32)
              + jnp.dot(hb, w1b[...], preferred_element_type=jnp.float32))


def _relu_body(q0, q1, o):
    o[...] = jnp.maximum(q0[...] + q1[...], 0.0)


def _softmax_body(r0, r1, o):
    ncls = o.shape[-1]
    s = r0[...] + r1[...]
    lane = lax.broadcasted_iota(jnp.int32, s.shape, 1)
    s = jnp.where(lane < ncls, s, -jnp.inf)
    m = jnp.max(s, axis=-1, keepdims=True)
    e = jnp.exp(s - m)
    p = e / jnp.sum(e, axis=-1, keepdims=True)
    o[...] = p[:, :ncls]


def kernel(x, edge_index, W0, W1, W2):
    n, d_feat = x.shape
    e = edge_index.shape[1]
    c0 = W0.shape[1]
    c1 = W1.shape[1]
    ncls = W2.shape[1]

    npad = ((n + 127) // 128) * 128
    ntiles = _NC * _NS
    nct = -(-e // (ntiles * _K * 16)) * 16
    epad = ntiles * nct * _K
    src_p = jnp.concatenate(
        [edge_index[0], jnp.zeros((epad - e,), jnp.int32)])
    dst_p = jnp.concatenate(
        [edge_index[1], jnp.full((epad - e,), npad - 1, jnp.int32)])
    idx2d = jnp.concatenate(
        [src_p.reshape(epad // _K, _K), dst_p.reshape(epad // _K, _K)], axis=0)
    zeros_f = jnp.zeros((npad, c1), jnp.float32)

    blk = 1000
    grid = (n // blk,)

    def mm(a, w):
        return pl.pallas_call(
            _mm_body,
            grid=grid,
            in_specs=[
                pl.BlockSpec((blk, a.shape[1]), lambda i: (i, 0)),
                pl.BlockSpec(w.shape, lambda i: (0, 0)),
            ],
            out_specs=pl.BlockSpec((blk, w.shape[1]), lambda i: (i, 0)),
            out_shape=jax.ShapeDtypeStruct((n, w.shape[1]), jnp.float32),
        )(a, w)

    h0a = mm(x, W0[:, :c1])
    h0b = mm(x, W0[:, c1:])
    pa = _spmm_partials(h0a, idx2d, zeros_f, n_nodes=n, channels=c1)
    pb = _spmm_partials(h0b, idx2d, zeros_f, n_nodes=n, channels=c1)

    h1p = pl.pallas_call(
        _l1_body,
        grid=grid,
        in_specs=[
            pl.BlockSpec((blk, c1), lambda i: (i, 0)),
            pl.BlockSpec((blk, c1), lambda i: (i, 0)),
            pl.BlockSpec((blk, c1), lambda i: (i, 0)),
            pl.BlockSpec((blk, c1), lambda i: (i, 0)),
            pl.BlockSpec((c1, c1), lambda i: (0, 0)),
            pl.BlockSpec((c1, c1), lambda i: (0, 0)),
        ],
        out_specs=pl.BlockSpec((blk, c1), lambda i: (i, 0)),
        out_shape=jax.ShapeDtypeStruct((n, c1), jnp.float32),
    )(pa[:n], pa[npad:npad + n], pb[:n], pb[npad:npad + n],
      W1[:c1], W1[c1:])

    q = _spmm_partials(h1p, idx2d, zeros_f, n_nodes=n, channels=c1)

    h1 = pl.pallas_call(
        _relu_body,
        grid=grid,
        in_specs=[
            pl.BlockSpec((blk, c1), lambda i: (i, 0)),
            pl.BlockSpec((blk, c1), lambda i: (i, 0)),
        ],
        out_specs=pl.BlockSpec((blk, c1), lambda i: (i, 0)),
        out_shape=jax.ShapeDtypeStruct((n, c1), jnp.float32),
    )(q[:n], q[npad:npad + n])

    w2p = jnp.pad(W2, ((0, 0), (0, 128 - ncls)))
    h2p = mm(h1, w2p)
    r = _spmm_partials(h2p, idx2d, zeros_f, n_nodes=n, channels=c1)

    out = pl.pallas_call(
        _softmax_body,
        grid=grid,
        in_specs=[
            pl.BlockSpec((blk, c1), lambda i: (i, 0)),
            pl.BlockSpec((blk, c1), lambda i: (i, 0)),
        ],
        out_specs=pl.BlockSpec((blk, ncls), lambda i: (i, 0)),
        out_shape=jax.ShapeDtypeStruct((n, ncls), jnp.float32),
    )(r[:n], r[npad:npad + n])

    return out
